# baseline (speedup 1.0000x reference)
import functools

import jax
import jax.numpy as jnp
from jax import lax
from jax.experimental import pallas as pl
from jax.experimental.pallas import tpu as pltpu
from jax.experimental.pallas import tpu_sc as plsc

N = 100000
E = 1600000
G = 16
NP = 100352
NT = NP // 16
EP = 1638400
EROWS = EP // 128
RT = EROWS // 16
NSB = 50
SBROWS = RT // NSB
NBUF = 4
CS = 2
TRIPS = SBROWS // (NBUF * CS)


def _make_agg(num_groups, table_rows):
    npasses = num_groups // 2
    mesh = plsc.VectorSubcoreMesh(core_axis_name="c", subcore_axis_name="s")

    @functools.partial(
        pl.kernel,
        out_type=jax.ShapeDtypeStruct((NP, num_groups, 16), jnp.float32),
        mesh=mesh,
        scratch_types=(
            [pltpu.VMEM((SBROWS, 2, 128), jnp.int32)
             for _ in range(2)]
            + [pltpu.VMEM((CS * 128, 16), jnp.float32)
               for _ in range(NBUF)]
            + [pltpu.VMEM_SHARED((NP, 16), jnp.float32)]
            + [pltpu.SemaphoreType.DMA for _ in range(2 * NBUF + 2)]
        ),
        compiler_params=pltpu.CompilerParams(use_tc_tiling_on_sc=False),
    )
    def agg(*refs):
        table = refs[0]
        idxcs = refs[1:1 + num_groups]
        zeros_hbm = refs[1 + num_groups]
        out = refs[2 + num_groups]
        sc = refs[3 + num_groups:]
        isets = sc[0:2]
        rows = sc[2:2 + NBUF]
        acc = sc[2 + NBUF]
        gsems = sc[3 + NBUF:3 + 2 * NBUF]
        ssems = sc[3 + 2 * NBUF:3 + 3 * NBUF]
        isems = sc[3 + 3 * NBUF:5 + 3 * NBUF]

        c = lax.axis_index("c")
        s = lax.axis_index("s")
        lo = s * NT

        def drain(b):
            pltpu.make_async_copy(
                zeros_hbm.at[pl.ds(0, CS * 128)], rows[b], ssems[b]).wait()

        def one_pass(group):
            idxg = idxcs[group]
            pltpu.sync_copy(zeros_hbm.at[pl.ds(lo, NT)], acc.at[pl.ds(lo, NT)])
            plsc.subcore_barrier()

            def stage(sb, p):
                base_row = s * RT + sb * SBROWS
                pltpu.async_copy(idxg.at[pl.ds(base_row, SBROWS)],
                                 isets[p], isems[p])

            def trips(p):
                iset = isets[p]

                def trip(j, carry):
                    r = j * NBUF * CS
                    hs = []
                    for b in range(NBUF):
                        @pl.when(j > 0)
                        def _(b=b):
                            drain(b)
                        hs.append([
                            pltpu.async_copy(
                                table.at[iset.at[r + b * CS + k, 0]],
                                rows[b].at[pl.ds(k * 128, 128)], gsems[b])
                            for k in range(CS)
                        ])
                    for b in range(NBUF):
                        for h in hs[b]:
                            h.wait()
                        for k in range(CS):
                            pltpu.async_copy(
                                rows[b].at[pl.ds(k * 128, 128)],
                                acc.at[iset.at[r + b * CS + k, 1]],
                                ssems[b], add=True)
                    return carry

                lax.fori_loop(0, TRIPS, trip, 0)

            def wait_iset(p):
                pltpu.make_async_copy(idxg.at[pl.ds(0, SBROWS)],
                                      isets[p], isems[p]).wait()

            stage(0, 0)

            def pair(i, carry2):
                @pl.when(i > 0)
                def _():
                    for b in range(NBUF):
                        drain(b)
                stage(2 * i + 1, 1)
                wait_iset(0)
                trips(0)
                for b in range(NBUF):
                    drain(b)

                @pl.when(i < NSB // 2 - 1)
                def _():
                    stage(2 * i + 2, 0)
                wait_iset(1)
                trips(1)
                return carry2

            lax.fori_loop(0, NSB // 2, pair, 0)
            for b in range(NBUF):
                drain(b)
            plsc.subcore_barrier()
            pltpu.sync_copy(acc.at[pl.ds(lo, NT)],
                            out.at[pl.ds(lo, NT), group])

        def run(groups):
            for g in groups:
                one_pass(g)

        pl.when(c == 0)(lambda: run(range(npasses)))
        pl.when(c == 1)(lambda: run(range(npasses, num_groups)))

    return agg


@functools.cache
def _agg(num_groups, table_rows):
    return _make_agg(num_groups, table_rows)


RB = 4000


def _mlp1_body(h0p, agg0, oh4, W1abd, b1abd, W1bbd, b1bbd, h1p, p0, p1):
    h0 = h0p[...]
    x = h0 + agg0[...]
    t = jnp.maximum(jnp.dot(x, W1abd[...], preferred_element_type=jnp.float32)
                    + b1abd[...], 0.0)
    h1 = jnp.maximum(jnp.dot(t, W1bbd[...], preferred_element_type=jnp.float32)
                     + b1bbd[...], 0.0)
    h1p[:, 0:1, :] = h1[:, 0:128].reshape(RB // 4, 1, 128)
    h1p[:, 1:2, :] = h1[:, 128:256].reshape(RB // 4, 1, 128)
    ohb = oh4[0]
    m0 = lax.dot_general(ohb, h0, (((0,), (0,)), ((), ())),
                         preferred_element_type=jnp.float32)
    m1 = lax.dot_general(ohb, h1, (((0,), (0,)), ((), ())),
                         preferred_element_type=jnp.float32)
    pp0 = sum(m0[16 * j:16 * (j + 1), 32 * j:32 * (j + 1)] for j in range(4))
    pp1 = sum(m1[16 * j:16 * (j + 1), 64 * j:64 * (j + 1)] for j in range(4))

    @pl.when(pl.program_id(0) == 0)
    def _():
        p0[...] = pp0
        p1[...] = pp1

    @pl.when(pl.program_id(0) != 0)
    def _():
        p0[...] += pp0
        p1[...] += pp1


def _mlp1(h0p, agg0v, oh4, W1abd, b1abd, W1bbd, b1bbd):
    grid = (N // RB,)
    return pl.pallas_call(
        _mlp1_body,
        grid=grid,
        in_specs=[
            pl.BlockSpec((RB // 4, 128), lambda i: (i, 0)),
            pl.BlockSpec((RB // 4, 128), lambda i: (i, 0)),
            pl.BlockSpec((1, RB // 4, 64), lambda i: (i, 0, 0)),
            pl.BlockSpec((128, 256), lambda i: (0, 0)),
            pl.BlockSpec((1, 256), lambda i: (0, 0)),
            pl.BlockSpec((256, 256), lambda i: (0, 0)),
            pl.BlockSpec((1, 256), lambda i: (0, 0)),
        ],
        out_specs=[
            pl.BlockSpec((RB // 4, 2, 128), lambda i: (i, 0, 0)),
            pl.BlockSpec((G, 32), lambda i: (0, 0)),
            pl.BlockSpec((G, 64), lambda i: (0, 0)),
        ],
        out_shape=[
            jax.ShapeDtypeStruct((N // 4, 2, 128), jnp.float32),
            jax.ShapeDtypeStruct((G, 32), jnp.float32),
            jax.ShapeDtypeStruct((G, 64), jnp.float32),
        ],
    )(h0p, agg0v, oh4, W1abd, b1abd, W1bbd, b1bbd)


def _mlp2_body(h1pv, agg1, oh2, W2abd, b2abd, W2bbd, b2bbd, p2):
    x = h1pv[...] + agg1[...]
    t = jnp.maximum(jnp.dot(x, W2abd[...], preferred_element_type=jnp.float32)
                    + b2abd[...], 0.0)
    h2 = jnp.maximum(jnp.dot(t, W2bbd[...], preferred_element_type=jnp.float32)
                     + b2bbd[...], 0.0)
    m2 = lax.dot_general(oh2[0], h2, (((0,), (0,)), ((), ())),
                         preferred_element_type=jnp.float32)
    pp2 = sum(m2[16 * j:16 * (j + 1), 64 * j:64 * (j + 1)] for j in range(2))

    @pl.when(pl.program_id(0) == 0)
    def _():
        p2[...] = pp2

    @pl.when(pl.program_id(0) != 0)
    def _():
        p2[...] += pp2


def _mlp2(h1pv, agg1v, oh2, W2abd, b2abd, W2bbd, b2bbd):
    grid = (N // RB,)
    return pl.pallas_call(
        _mlp2_body,
        grid=grid,
        in_specs=[
            pl.BlockSpec((RB // 2, 128), lambda i: (i, 0)),
            pl.BlockSpec((RB // 2, 128), lambda i: (i, 0)),
            pl.BlockSpec((1, RB // 2, 32), lambda i: (i, 0, 0)),
            pl.BlockSpec((128, 128), lambda i: (0, 0)),
            pl.BlockSpec((1, 128), lambda i: (0, 0)),
            pl.BlockSpec((128, 128), lambda i: (0, 0)),
            pl.BlockSpec((1, 128), lambda i: (0, 0)),
        ],
        out_specs=pl.BlockSpec((G, 64), lambda i: (0, 0)),
        out_shape=jax.ShapeDtypeStruct((G, 64), jnp.float32),
    )(h1pv, agg1v, oh2, W2abd, b2abd, W2bbd, b2bbd)


def _heads_body(p0, p1, p2, Wp0, bp0, Wp1, bp1, Wp2, bp2,
                Wm1, bm1, Wm2, bm2, Wmean, bmean, Wstd, bstd, mean, std):
    score = (jnp.dot(p0[...], Wp0[...], preferred_element_type=jnp.float32)
             + bp0[...]
             + jnp.dot(p1[...], Wp1[...], preferred_element_type=jnp.float32)
             + bp1[...]
             + jnp.dot(p2[...], Wp2[...], preferred_element_type=jnp.float32)
             + bp2[...])
    f = jnp.maximum(jnp.dot(score, Wm1[...], preferred_element_type=jnp.float32)
                    + bm1[...], 0.0)
    f = jnp.maximum(jnp.dot(f, Wm2[...], preferred_element_type=jnp.float32)
                    + bm2[...], 0.0)
    mean[...] = jnp.dot(f, Wmean[...], preferred_element_type=jnp.float32) \
        + bmean[...]
    z = jnp.dot(f, Wstd[...], preferred_element_type=jnp.float32) + bstd[...]
    std[...] = jnp.maximum(z, 0.0) + jnp.log1p(jnp.exp(-jnp.abs(z)))


def _heads(p0, p1, p2, Wp0, bp0, Wp1, bp1, Wp2, bp2,
           Wm1, bm1, Wm2, bm2, Wmean, bmean, Wstd, bstd):
    return pl.pallas_call(
        _heads_body,
        out_shape=[
            jax.ShapeDtypeStruct((G, 32), jnp.float32),
            jax.ShapeDtypeStruct((G, 32), jnp.float32),
        ],
    )(p0, p1, p2, Wp0, bp0, Wp1, bp1, Wp2, bp2,
      Wm1, bm1, Wm2, bm2, Wmean, bmean, Wstd, bstd)


def kernel(node_feats, edge_index, graph_ids,
           W1a, b1a, W1b, b1b, W2a, b2a, W2b, b2b,
           Wp0, bp0, Wp1, bp1, Wp2, bp2,
           Wm1, bm1, Wm2, bm2, Wmean, bmean, Wstd, bstd):
    npad = EP - E
    srcp = jnp.concatenate(
        [edge_index[0], jnp.arange(npad, dtype=jnp.int32) % N])
    dst = jnp.concatenate(
        [edge_index[1],
         N + (jnp.arange(npad, dtype=jnp.int32) % (NP - N))]) \
        .reshape(EROWS, 128)
    zeros = jnp.zeros((NP // 8, 128), jnp.float32).reshape(NP, 16)
    oh = (graph_ids[:, None] == jnp.arange(G, dtype=jnp.int32)[None, :]) \
        .astype(jnp.float32)
    oh4 = oh.reshape(N // RB, RB // 4, 64)
    oh2 = oh.reshape(N // RB, RB // 2, 32)

    t0 = node_feats.reshape(8 * N, 16)
    i0 = [jnp.stack([(8 * srcp + k).reshape(EROWS, 128), dst], axis=1)
          for k in range(2)]
    agg0 = _agg(2, 8 * N)(t0, i0[0], i0[1], zeros)
    agg0v = agg0.reshape(NP // 4, 128)

    h0p = node_feats[:, 0:32].reshape(N // 4, 128)
    bd = jax.scipy.linalg.block_diag
    h1p, p0, p1 = _mlp1(
        h0p, agg0v, oh4,
        bd(W1a, W1a, W1a, W1a), jnp.tile(b1a, 4).reshape(1, 256),
        bd(W1b, W1b, W1b, W1b), jnp.tile(b1b, 4).reshape(1, 256))

    t1 = h1p.reshape(4 * N, 16)
    i1 = [jnp.stack([(4 * srcp + k).reshape(EROWS, 128), dst], axis=1)
          for k in range(4)]
    agg1 = _agg(4, 4 * N)(t1, i1[0], i1[1], i1[2], i1[3], zeros)
    agg1v = agg1.reshape(NP // 2, 128)

    h1pv = h1p.reshape(N // 2, 128)
    p2 = _mlp2(h1pv, agg1v, oh2,
               bd(W2a, W2a), jnp.tile(b2a, 2).reshape(1, 128),
               bd(W2b, W2b), jnp.tile(b2b, 2).reshape(1, 128))
    mean, std = _heads(
        p0, p1, p2, Wp0, bp0.reshape(1, -1), Wp1, bp1.reshape(1, -1),
        Wp2, bp2.reshape(1, -1), Wm1, bm1.reshape(1, -1),
        Wm2, bm2.reshape(1, -1), Wmean, bmean.reshape(1, -1),
        Wstd, bstd.reshape(1, -1))
    return mean, std


# speedup vs baseline: 1.0134x; 1.0134x over previous
import functools

import jax
import jax.numpy as jnp
from jax import lax
from jax.experimental import pallas as pl
from jax.experimental.pallas import tpu as pltpu
from jax.experimental.pallas import tpu_sc as plsc

N = 100000
E = 1600000
G = 16
NP = 100352
NT = NP // 16
EP = 1638400
EROWS = EP // 128
RT = EROWS // 16
NSB = 20
SBROWS = RT // NSB
NBUF = 4
CS = 2
TRIPS = SBROWS // (NBUF * CS)


def _make_agg(num_groups, table_rows):
    npasses = num_groups // 2
    mesh = plsc.VectorSubcoreMesh(core_axis_name="c", subcore_axis_name="s")

    @functools.partial(
        pl.kernel,
        out_type=jax.ShapeDtypeStruct((NP, num_groups, 16), jnp.float32),
        mesh=mesh,
        scratch_types=(
            [pltpu.VMEM((SBROWS, 128), jnp.int32),
             pltpu.VMEM((SBROWS, 128), jnp.int32)]
            + [pltpu.VMEM((CS * 128, 16), jnp.float32)
               for _ in range(NBUF)]
            + [pltpu.VMEM_SHARED((NP, 16), jnp.float32)]
            + [pltpu.SemaphoreType.DMA for _ in range(2 * NBUF)]
        ),
        compiler_params=pltpu.CompilerParams(use_tc_tiling_on_sc=False),
    )
    def agg(*refs):
        table = refs[0]
        idxs = refs[1:1 + num_groups]
        dstg, zeros_hbm = refs[1 + num_groups:3 + num_groups]
        out = refs[3 + num_groups]
        isrc, idst = refs[4 + num_groups:6 + num_groups]
        rows = refs[6 + num_groups:6 + num_groups + NBUF]
        acc = refs[6 + num_groups + NBUF]
        gsems = refs[7 + num_groups + NBUF:7 + num_groups + 2 * NBUF]
        ssems = refs[7 + num_groups + 2 * NBUF:7 + num_groups + 3 * NBUF]

        c = lax.axis_index("c")
        s = lax.axis_index("s")
        lo = s * NT

        def drain(b):
            pltpu.make_async_copy(
                zeros_hbm.at[pl.ds(0, CS * 128)], rows[b], ssems[b]).wait()

        def one_pass(group):
            idxg = idxs[group]
            pltpu.sync_copy(zeros_hbm.at[pl.ds(lo, NT)], acc.at[pl.ds(lo, NT)])
            plsc.subcore_barrier()

            def superblock(sb, carry2):
                @pl.when(sb > 0)
                def _():
                    for b in range(NBUF):
                        drain(b)
                base_row = s * RT + sb * SBROWS
                pltpu.sync_copy(idxg.at[pl.ds(base_row, SBROWS)], isrc)
                pltpu.sync_copy(dstg.at[pl.ds(base_row, SBROWS)], idst)

                def trip(j, carry):
                    r = j * NBUF * CS
                    hs = []
                    for b in range(NBUF):
                        @pl.when(j > 0)
                        def _(b=b):
                            drain(b)
                        hs.append([
                            pltpu.async_copy(
                                table.at[isrc.at[r + b * CS + k]],
                                rows[b].at[pl.ds(k * 128, 128)], gsems[b])
                            for k in range(CS)
                        ])
                    for b in range(NBUF):
                        for h in hs[b]:
                            h.wait()
                        for k in range(CS):
                            pltpu.async_copy(
                                rows[b].at[pl.ds(k * 128, 128)],
                                acc.at[idst.at[r + b * CS + k]],
                                ssems[b], add=True)
                    return carry

                lax.fori_loop(0, TRIPS, trip, 0)
                return carry2

            lax.fori_loop(0, NSB, superblock, 0)
            for b in range(NBUF):
                drain(b)
            plsc.subcore_barrier()
            pltpu.sync_copy(acc.at[pl.ds(lo, NT)],
                            out.at[pl.ds(lo, NT), group])

        def run(groups):
            for g in groups:
                one_pass(g)

        pl.when(c == 0)(lambda: run(range(npasses)))
        pl.when(c == 1)(lambda: run(range(npasses, num_groups)))

    return agg


@functools.cache
def _agg(num_groups, table_rows):
    return _make_agg(num_groups, table_rows)


RB = 4000


def _mlp1_body(h0p, agg0, oh4, W1abd, b1abd, W1bbd, b1bbd, h1p, p0, p1):
    h0 = h0p[...]
    x = h0 + agg0[...]
    t = jnp.maximum(jnp.dot(x, W1abd[...], preferred_element_type=jnp.float32)
                    + b1abd[...], 0.0)
    h1 = jnp.maximum(jnp.dot(t, W1bbd[...], preferred_element_type=jnp.float32)
                     + b1bbd[...], 0.0)
    h1p[:, 0:1, :] = h1[:, 0:128].reshape(RB // 4, 1, 128)
    h1p[:, 1:2, :] = h1[:, 128:256].reshape(RB // 4, 1, 128)
    ohb = oh4[0]
    m0 = lax.dot_general(ohb, h0, (((0,), (0,)), ((), ())),
                         preferred_element_type=jnp.float32)
    m1 = lax.dot_general(ohb, h1, (((0,), (0,)), ((), ())),
                         preferred_element_type=jnp.float32)
    pp0 = sum(m0[16 * j:16 * (j + 1), 32 * j:32 * (j + 1)] for j in range(4))
    pp1 = sum(m1[16 * j:16 * (j + 1), 64 * j:64 * (j + 1)] for j in range(4))

    @pl.when(pl.program_id(0) == 0)
    def _():
        p0[...] = pp0
        p1[...] = pp1

    @pl.when(pl.program_id(0) != 0)
    def _():
        p0[...] += pp0
        p1[...] += pp1


def _mlp1(h0p, agg0v, oh4, W1abd, b1abd, W1bbd, b1bbd):
    grid = (N // RB,)
    return pl.pallas_call(
        _mlp1_body,
        grid=grid,
        in_specs=[
            pl.BlockSpec((RB // 4, 128), lambda i: (i, 0)),
            pl.BlockSpec((RB // 4, 128), lambda i: (i, 0)),
            pl.BlockSpec((1, RB // 4, 64), lambda i: (i, 0, 0)),
            pl.BlockSpec((128, 256), lambda i: (0, 0)),
            pl.BlockSpec((1, 256), lambda i: (0, 0)),
            pl.BlockSpec((256, 256), lambda i: (0, 0)),
            pl.BlockSpec((1, 256), lambda i: (0, 0)),
        ],
        out_specs=[
            pl.BlockSpec((RB // 4, 2, 128), lambda i: (i, 0, 0)),
            pl.BlockSpec((G, 32), lambda i: (0, 0)),
            pl.BlockSpec((G, 64), lambda i: (0, 0)),
        ],
        out_shape=[
            jax.ShapeDtypeStruct((N // 4, 2, 128), jnp.float32),
            jax.ShapeDtypeStruct((G, 32), jnp.float32),
            jax.ShapeDtypeStruct((G, 64), jnp.float32),
        ],
    )(h0p, agg0v, oh4, W1abd, b1abd, W1bbd, b1bbd)


def _mlp2_body(h1pv, agg1, oh2, W2abd, b2abd, W2bbd, b2bbd, p2):
    x = h1pv[...] + agg1[...]
    t = jnp.maximum(jnp.dot(x, W2abd[...], preferred_element_type=jnp.float32)
                    + b2abd[...], 0.0)
    h2 = jnp.maximum(jnp.dot(t, W2bbd[...], preferred_element_type=jnp.float32)
                     + b2bbd[...], 0.0)
    m2 = lax.dot_general(oh2[0], h2, (((0,), (0,)), ((), ())),
                         preferred_element_type=jnp.float32)
    pp2 = sum(m2[16 * j:16 * (j + 1), 64 * j:64 * (j + 1)] for j in range(2))

    @pl.when(pl.program_id(0) == 0)
    def _():
        p2[...] = pp2

    @pl.when(pl.program_id(0) != 0)
    def _():
        p2[...] += pp2


def _mlp2(h1pv, agg1v, oh2, W2abd, b2abd, W2bbd, b2bbd):
    grid = (N // RB,)
    return pl.pallas_call(
        _mlp2_body,
        grid=grid,
        in_specs=[
            pl.BlockSpec((RB // 2, 128), lambda i: (i, 0)),
            pl.BlockSpec((RB // 2, 128), lambda i: (i, 0)),
            pl.BlockSpec((1, RB // 2, 32), lambda i: (i, 0, 0)),
            pl.BlockSpec((128, 128), lambda i: (0, 0)),
            pl.BlockSpec((1, 128), lambda i: (0, 0)),
            pl.BlockSpec((128, 128), lambda i: (0, 0)),
            pl.BlockSpec((1, 128), lambda i: (0, 0)),
        ],
        out_specs=pl.BlockSpec((G, 64), lambda i: (0, 0)),
        out_shape=jax.ShapeDtypeStruct((G, 64), jnp.float32),
    )(h1pv, agg1v, oh2, W2abd, b2abd, W2bbd, b2bbd)


def _heads_body(p0, p1, p2, Wp0, bp0, Wp1, bp1, Wp2, bp2,
                Wm1, bm1, Wm2, bm2, Wmean, bmean, Wstd, bstd, mean, std):
    score = (jnp.dot(p0[...], Wp0[...], preferred_element_type=jnp.float32)
             + bp0[...]
             + jnp.dot(p1[...], Wp1[...], preferred_element_type=jnp.float32)
             + bp1[...]
             + jnp.dot(p2[...], Wp2[...], preferred_element_type=jnp.float32)
             + bp2[...])
    f = jnp.maximum(jnp.dot(score, Wm1[...], preferred_element_type=jnp.float32)
                    + bm1[...], 0.0)
    f = jnp.maximum(jnp.dot(f, Wm2[...], preferred_element_type=jnp.float32)
                    + bm2[...], 0.0)
    mean[...] = jnp.dot(f, Wmean[...], preferred_element_type=jnp.float32) \
        + bmean[...]
    z = jnp.dot(f, Wstd[...], preferred_element_type=jnp.float32) + bstd[...]
    std[...] = jnp.maximum(z, 0.0) + jnp.log1p(jnp.exp(-jnp.abs(z)))


def _heads(p0, p1, p2, Wp0, bp0, Wp1, bp1, Wp2, bp2,
           Wm1, bm1, Wm2, bm2, Wmean, bmean, Wstd, bstd):
    return pl.pallas_call(
        _heads_body,
        out_shape=[
            jax.ShapeDtypeStruct((G, 32), jnp.float32),
            jax.ShapeDtypeStruct((G, 32), jnp.float32),
        ],
    )(p0, p1, p2, Wp0, bp0, Wp1, bp1, Wp2, bp2,
      Wm1, bm1, Wm2, bm2, Wmean, bmean, Wstd, bstd)


def kernel(node_feats, edge_index, graph_ids,
           W1a, b1a, W1b, b1b, W2a, b2a, W2b, b2b,
           Wp0, bp0, Wp1, bp1, Wp2, bp2,
           Wm1, bm1, Wm2, bm2, Wmean, bmean, Wstd, bstd):
    npad = EP - E
    srcp = jnp.concatenate(
        [edge_index[0], jnp.arange(npad, dtype=jnp.int32) % N])
    dst = jnp.concatenate(
        [edge_index[1],
         N + (jnp.arange(npad, dtype=jnp.int32) % (NP - N))]) \
        .reshape(EROWS, 128)
    zeros = jnp.zeros((NP // 8, 128), jnp.float32).reshape(NP, 16)
    oh = (graph_ids[:, None] == jnp.arange(G, dtype=jnp.int32)[None, :]) \
        .astype(jnp.float32)
    oh4 = oh.reshape(N // RB, RB // 4, 64)
    oh2 = oh.reshape(N // RB, RB // 2, 32)

    t0 = node_feats.reshape(8 * N, 16)
    i0 = [(8 * srcp + k).reshape(EROWS, 128) for k in range(2)]
    agg0 = _agg(2, 8 * N)(t0, i0[0], i0[1], dst, zeros)
    agg0v = agg0.reshape(NP // 4, 128)

    h0p = node_feats[:, 0:32].reshape(N // 4, 128)
    bd = jax.scipy.linalg.block_diag
    h1p, p0, p1 = _mlp1(
        h0p, agg0v, oh4,
        bd(W1a, W1a, W1a, W1a), jnp.tile(b1a, 4).reshape(1, 256),
        bd(W1b, W1b, W1b, W1b), jnp.tile(b1b, 4).reshape(1, 256))

    t1 = h1p.reshape(4 * N, 16)
    i1 = [(4 * srcp + k).reshape(EROWS, 128) for k in range(4)]
    agg1 = _agg(4, 4 * N)(t1, i1[0], i1[1], i1[2], i1[3], dst, zeros)
    agg1v = agg1.reshape(NP // 2, 128)

    h1pv = h1p.reshape(N // 2, 128)
    p2 = _mlp2(h1pv, agg1v, oh2,
               bd(W2a, W2a), jnp.tile(b2a, 2).reshape(1, 128),
               bd(W2b, W2b), jnp.tile(b2b, 2).reshape(1, 128))
    mean, std = _heads(
        p0, p1, p2, Wp0, bp0.reshape(1, -1), Wp1, bp1.reshape(1, -1),
        Wp2, bp2.reshape(1, -1), Wm1, bm1.reshape(1, -1),
        Wm2, bm2.reshape(1, -1), Wmean, bmean.reshape(1, -1),
        Wstd, bstd.reshape(1, -1))
    return mean, std


# speedup vs baseline: 1.0616x; 1.0475x over previous
import functools

import jax
import jax.numpy as jnp
from jax import lax
from jax.experimental import pallas as pl
from jax.experimental.pallas import tpu as pltpu
from jax.experimental.pallas import tpu_sc as plsc

N = 100000
E = 1600000
G = 16
NP = 100224
NT = NP // 16
EP = 1638400
EROWS = EP // 128
RT = EROWS // 16
NSB = 20
SBROWS = RT // NSB
NBUF = 5
CS = 2
TRIPS = SBROWS // (NBUF * CS)


def _make_agg(num_groups, table_rows):
    npasses = num_groups // 2
    mesh = plsc.VectorSubcoreMesh(core_axis_name="c", subcore_axis_name="s")

    @functools.partial(
        pl.kernel,
        out_type=jax.ShapeDtypeStruct((NP, num_groups, 16), jnp.float32),
        mesh=mesh,
        scratch_types=(
            [pltpu.VMEM((SBROWS, 128), jnp.int32),
             pltpu.VMEM((SBROWS, 128), jnp.int32)]
            + [pltpu.VMEM((CS * 128, 16), jnp.float32)
               for _ in range(NBUF)]
            + [pltpu.VMEM_SHARED((NP, 16), jnp.float32)]
            + [pltpu.SemaphoreType.DMA for _ in range(2 * NBUF)]
        ),
        compiler_params=pltpu.CompilerParams(use_tc_tiling_on_sc=False),
    )
    def agg(*refs):
        table = refs[0]
        idxs = refs[1:1 + num_groups]
        dstg, zeros_hbm = refs[1 + num_groups:3 + num_groups]
        out = refs[3 + num_groups]
        isrc, idst = refs[4 + num_groups:6 + num_groups]
        rows = refs[6 + num_groups:6 + num_groups + NBUF]
        acc = refs[6 + num_groups + NBUF]
        gsems = refs[7 + num_groups + NBUF:7 + num_groups + 2 * NBUF]
        ssems = refs[7 + num_groups + 2 * NBUF:7 + num_groups + 3 * NBUF]

        c = lax.axis_index("c")
        s = lax.axis_index("s")
        lo = s * NT

        def drain(b):
            pltpu.make_async_copy(
                zeros_hbm.at[pl.ds(0, CS * 128)], rows[b], ssems[b]).wait()

        def one_pass(group):
            idxg = idxs[group]
            pltpu.sync_copy(zeros_hbm.at[pl.ds(lo, NT)], acc.at[pl.ds(lo, NT)])
            plsc.subcore_barrier()

            def superblock(sb, carry2):
                @pl.when(sb > 0)
                def _():
                    for b in range(NBUF):
                        drain(b)
                base_row = s * RT + sb * SBROWS
                pltpu.sync_copy(idxg.at[pl.ds(base_row, SBROWS)], isrc)
                pltpu.sync_copy(dstg.at[pl.ds(base_row, SBROWS)], idst)

                def trip(j, carry):
                    r = j * NBUF * CS
                    hs = []
                    for b in range(NBUF):
                        @pl.when(j > 0)
                        def _(b=b):
                            drain(b)
                        hs.append([
                            pltpu.async_copy(
                                table.at[isrc.at[r + b * CS + k]],
                                rows[b].at[pl.ds(k * 128, 128)], gsems[b])
                            for k in range(CS)
                        ])
                    for b in range(NBUF):
                        for h in hs[b]:
                            h.wait()
                        for k in range(CS):
                            pltpu.async_copy(
                                rows[b].at[pl.ds(k * 128, 128)],
                                acc.at[idst.at[r + b * CS + k]],
                                ssems[b], add=True)
                    return carry

                lax.fori_loop(0, TRIPS, trip, 0)
                return carry2

            lax.fori_loop(0, NSB, superblock, 0)
            for b in range(NBUF):
                drain(b)
            plsc.subcore_barrier()
            pltpu.sync_copy(acc.at[pl.ds(lo, NT)],
                            out.at[pl.ds(lo, NT), group])

        def run(groups):
            for g in groups:
                one_pass(g)

        pl.when(c == 0)(lambda: run(range(npasses)))
        pl.when(c == 1)(lambda: run(range(npasses, num_groups)))

    return agg


@functools.cache
def _agg(num_groups, table_rows):
    return _make_agg(num_groups, table_rows)


RB = 4000


def _mlp1_body(h0p, agg0, oh4, W1abd, b1abd, W1bbd, b1bbd, h1p, p0, p1):
    h0 = h0p[...]
    x = h0 + agg0[...]
    t = jnp.maximum(jnp.dot(x, W1abd[...], preferred_element_type=jnp.float32)
                    + b1abd[...], 0.0)
    h1 = jnp.maximum(jnp.dot(t, W1bbd[...], preferred_element_type=jnp.float32)
                     + b1bbd[...], 0.0)
    h1p[:, 0:1, :] = h1[:, 0:128].reshape(RB // 4, 1, 128)
    h1p[:, 1:2, :] = h1[:, 128:256].reshape(RB // 4, 1, 128)
    ohb = oh4[0]
    m0 = lax.dot_general(ohb, h0, (((0,), (0,)), ((), ())),
                         preferred_element_type=jnp.float32)
    m1 = lax.dot_general(ohb, h1, (((0,), (0,)), ((), ())),
                         preferred_element_type=jnp.float32)
    pp0 = sum(m0[16 * j:16 * (j + 1), 32 * j:32 * (j + 1)] for j in range(4))
    pp1 = sum(m1[16 * j:16 * (j + 1), 64 * j:64 * (j + 1)] for j in range(4))

    @pl.when(pl.program_id(0) == 0)
    def _():
        p0[...] = pp0
        p1[...] = pp1

    @pl.when(pl.program_id(0) != 0)
    def _():
        p0[...] += pp0
        p1[...] += pp1


def _mlp1(h0p, agg0v, oh4, W1abd, b1abd, W1bbd, b1bbd):
    grid = (N // RB,)
    return pl.pallas_call(
        _mlp1_body,
        grid=grid,
        in_specs=[
            pl.BlockSpec((RB // 4, 128), lambda i: (i, 0)),
            pl.BlockSpec((RB // 4, 128), lambda i: (i, 0)),
            pl.BlockSpec((1, RB // 4, 64), lambda i: (i, 0, 0)),
            pl.BlockSpec((128, 256), lambda i: (0, 0)),
            pl.BlockSpec((1, 256), lambda i: (0, 0)),
            pl.BlockSpec((256, 256), lambda i: (0, 0)),
            pl.BlockSpec((1, 256), lambda i: (0, 0)),
        ],
        out_specs=[
            pl.BlockSpec((RB // 4, 2, 128), lambda i: (i, 0, 0)),
            pl.BlockSpec((G, 32), lambda i: (0, 0)),
            pl.BlockSpec((G, 64), lambda i: (0, 0)),
        ],
        out_shape=[
            jax.ShapeDtypeStruct((N // 4, 2, 128), jnp.float32),
            jax.ShapeDtypeStruct((G, 32), jnp.float32),
            jax.ShapeDtypeStruct((G, 64), jnp.float32),
        ],
    )(h0p, agg0v, oh4, W1abd, b1abd, W1bbd, b1bbd)


def _mlp2_body(h1pv, agg1, oh2, W2abd, b2abd, W2bbd, b2bbd, p2):
    x = h1pv[...] + agg1[...]
    t = jnp.maximum(jnp.dot(x, W2abd[...], preferred_element_type=jnp.float32)
                    + b2abd[...], 0.0)
    h2 = jnp.maximum(jnp.dot(t, W2bbd[...], preferred_element_type=jnp.float32)
                     + b2bbd[...], 0.0)
    m2 = lax.dot_general(oh2[0], h2, (((0,), (0,)), ((), ())),
                         preferred_element_type=jnp.float32)
    pp2 = sum(m2[16 * j:16 * (j + 1), 64 * j:64 * (j + 1)] for j in range(2))

    @pl.when(pl.program_id(0) == 0)
    def _():
        p2[...] = pp2

    @pl.when(pl.program_id(0) != 0)
    def _():
        p2[...] += pp2


def _mlp2(h1pv, agg1v, oh2, W2abd, b2abd, W2bbd, b2bbd):
    grid = (N // RB,)
    return pl.pallas_call(
        _mlp2_body,
        grid=grid,
        in_specs=[
            pl.BlockSpec((RB // 2, 128), lambda i: (i, 0)),
            pl.BlockSpec((RB // 2, 128), lambda i: (i, 0)),
            pl.BlockSpec((1, RB // 2, 32), lambda i: (i, 0, 0)),
            pl.BlockSpec((128, 128), lambda i: (0, 0)),
            pl.BlockSpec((1, 128), lambda i: (0, 0)),
            pl.BlockSpec((128, 128), lambda i: (0, 0)),
            pl.BlockSpec((1, 128), lambda i: (0, 0)),
        ],
        out_specs=pl.BlockSpec((G, 64), lambda i: (0, 0)),
        out_shape=jax.ShapeDtypeStruct((G, 64), jnp.float32),
    )(h1pv, agg1v, oh2, W2abd, b2abd, W2bbd, b2bbd)


def _heads_body(p0, p1, p2, Wp0, bp0, Wp1, bp1, Wp2, bp2,
                Wm1, bm1, Wm2, bm2, Wmean, bmean, Wstd, bstd, mean, std):
    score = (jnp.dot(p0[...], Wp0[...], preferred_element_type=jnp.float32)
             + bp0[...]
             + jnp.dot(p1[...], Wp1[...], preferred_element_type=jnp.float32)
             + bp1[...]
             + jnp.dot(p2[...], Wp2[...], preferred_element_type=jnp.float32)
             + bp2[...])
    f = jnp.maximum(jnp.dot(score, Wm1[...], preferred_element_type=jnp.float32)
                    + bm1[...], 0.0)
    f = jnp.maximum(jnp.dot(f, Wm2[...], preferred_element_type=jnp.float32)
                    + bm2[...], 0.0)
    mean[...] = jnp.dot(f, Wmean[...], preferred_element_type=jnp.float32) \
        + bmean[...]
    z = jnp.dot(f, Wstd[...], preferred_element_type=jnp.float32) + bstd[...]
    std[...] = jnp.maximum(z, 0.0) + jnp.log1p(jnp.exp(-jnp.abs(z)))


def _heads(p0, p1, p2, Wp0, bp0, Wp1, bp1, Wp2, bp2,
           Wm1, bm1, Wm2, bm2, Wmean, bmean, Wstd, bstd):
    return pl.pallas_call(
        _heads_body,
        out_shape=[
            jax.ShapeDtypeStruct((G, 32), jnp.float32),
            jax.ShapeDtypeStruct((G, 32), jnp.float32),
        ],
    )(p0, p1, p2, Wp0, bp0, Wp1, bp1, Wp2, bp2,
      Wm1, bm1, Wm2, bm2, Wmean, bmean, Wstd, bstd)


def kernel(node_feats, edge_index, graph_ids,
           W1a, b1a, W1b, b1b, W2a, b2a, W2b, b2b,
           Wp0, bp0, Wp1, bp1, Wp2, bp2,
           Wm1, bm1, Wm2, bm2, Wmean, bmean, Wstd, bstd):
    npad = EP - E
    srcp = jnp.concatenate(
        [edge_index[0], jnp.arange(npad, dtype=jnp.int32) % N])
    dst = jnp.concatenate(
        [edge_index[1],
         N + (jnp.arange(npad, dtype=jnp.int32) % (NP - N))]) \
        .reshape(EROWS, 128)
    zeros = jnp.zeros((NP // 8, 128), jnp.float32).reshape(NP, 16)
    oh = (graph_ids[:, None] == jnp.arange(G, dtype=jnp.int32)[None, :]) \
        .astype(jnp.float32)
    oh4 = oh.reshape(N // RB, RB // 4, 64)
    oh2 = oh.reshape(N // RB, RB // 2, 32)

    t0 = node_feats.reshape(8 * N, 16)
    i0 = [(8 * srcp + k).reshape(EROWS, 128) for k in range(2)]
    agg0 = _agg(2, 8 * N)(t0, i0[0], i0[1], dst, zeros)
    agg0v = agg0.reshape(NP // 4, 128)

    h0p = node_feats[:, 0:32].reshape(N // 4, 128)
    bd = jax.scipy.linalg.block_diag
    h1p, p0, p1 = _mlp1(
        h0p, agg0v, oh4,
        bd(W1a, W1a, W1a, W1a), jnp.tile(b1a, 4).reshape(1, 256),
        bd(W1b, W1b, W1b, W1b), jnp.tile(b1b, 4).reshape(1, 256))

    t1 = h1p.reshape(4 * N, 16)
    i1 = [(4 * srcp + k).reshape(EROWS, 128) for k in range(4)]
    agg1 = _agg(4, 4 * N)(t1, i1[0], i1[1], i1[2], i1[3], dst, zeros)
    agg1v = agg1.reshape(NP // 2, 128)

    h1pv = h1p.reshape(N // 2, 128)
    p2 = _mlp2(h1pv, agg1v, oh2,
               bd(W2a, W2a), jnp.tile(b2a, 2).reshape(1, 128),
               bd(W2b, W2b), jnp.tile(b2b, 2).reshape(1, 128))
    mean, std = _heads(
        p0, p1, p2, Wp0, bp0.reshape(1, -1), Wp1, bp1.reshape(1, -1),
        Wp2, bp2.reshape(1, -1), Wm1, bm1.reshape(1, -1),
        Wm2, bm2.reshape(1, -1), Wmean, bmean.reshape(1, -1),
        Wstd, bstd.reshape(1, -1))
    return mean, std
